# baseline (speedup 1.0000x reference)
import jax
import jax.numpy as jnp
from jax.experimental import pallas as pl
from jax.experimental.pallas import tpu as pltpu

NODE_DIM = 512
HID_DIM = 2048
OUT_DIM = 512
N_TOKENS = 32768
NUM_SEGMENTS = 16

BLOCK_T = 2048
NUM_BLOCKS = N_TOKENS // BLOCK_T
N_CHUNKS = 4
CHUNK_T = BLOCK_T // N_CHUNKS


def _x_copy(x_hbm, xbuf, sem, blk, slot):
    return pltpu.make_async_copy(
        x_hbm.at[pl.ds(blk * BLOCK_T, BLOCK_T), :],
        xbuf.at[slot],
        sem.at[slot],
    )


def _mlpagg_kernel(seg_ref, x_hbm, W1_ref, b1_ref, W2_ref, b2_ref,
                   W3_ref, b3_ref, out_ref, xbuf, W1b, W2b, acc_ref,
                   cnt_ref, sem):
    i = pl.program_id(0)
    slot = jax.lax.rem(i, 2)

    @pl.when(i == 0)
    def _init():
        acc_ref[...] = jnp.zeros_like(acc_ref)
        cnt_ref[...] = jnp.zeros_like(cnt_ref)
        W1b[...] = W1_ref[...].astype(jnp.bfloat16)
        W2b[...] = W2_ref[...].astype(jnp.bfloat16)
        _x_copy(x_hbm, xbuf, sem, 0, 0).start()

    @pl.when(i + 1 < NUM_BLOCKS)
    def _prefetch():
        _x_copy(x_hbm, xbuf, sem, i + 1, 1 - slot).start()

    _x_copy(x_hbm, xbuf, sem, i, slot).wait()

    seg = seg_ref[0, 0, :]
    onehot = (seg[None, :] == jax.lax.broadcasted_iota(
        jnp.int32, (NUM_SEGMENTS, BLOCK_T), 0)).astype(jnp.bfloat16)
    cnt_ref[...] += jnp.sum(onehot.astype(jnp.float32), axis=1,
                            keepdims=True)

    acc = jnp.zeros((NUM_SEGMENTS, HID_DIM), dtype=jnp.float32)
    for c in range(N_CHUNKS):
        sl = slice(c * CHUNK_T, (c + 1) * CHUNK_T)
        x = xbuf[slot, sl, :].astype(jnp.bfloat16)
        h = jnp.dot(x, W1b[...], preferred_element_type=jnp.float32)
        h = jnp.maximum(h + b1_ref[...], 0.0).astype(jnp.bfloat16)
        h = jnp.dot(h, W2b[...], preferred_element_type=jnp.float32)
        h = jnp.maximum(h + b2_ref[...], 0.0).astype(jnp.bfloat16)
        acc = acc + jnp.dot(onehot[:, sl], h,
                            preferred_element_type=jnp.float32)
    acc_ref[...] += acc

    @pl.when(i == NUM_BLOCKS - 1)
    def _finish():
        counts = jnp.maximum(cnt_ref[:, 0:1], 1.0)
        mean = acc_ref[...] / counts
        out_ref[...] = jnp.dot(
            mean, W3_ref[...], preferred_element_type=jnp.float32) + b3_ref[...]


@jax.jit
def kernel(x, x_batch, W1, b1, W2, b2, W3, b3):
    seg = x_batch.astype(jnp.int32).reshape(NUM_BLOCKS, 1, BLOCK_T)
    b1 = b1.reshape(1, HID_DIM)
    b2 = b2.reshape(1, HID_DIM)
    b3 = b3.reshape(1, OUT_DIM)

    grid = (NUM_BLOCKS,)
    out = pl.pallas_call(
        _mlpagg_kernel,
        grid=grid,
        in_specs=[
            pl.BlockSpec((1, 1, BLOCK_T), lambda i: (i, 0, 0)),
            pl.BlockSpec(memory_space=pl.ANY),
            pl.BlockSpec((NODE_DIM, HID_DIM), lambda i: (0, 0)),
            pl.BlockSpec((1, HID_DIM), lambda i: (0, 0)),
            pl.BlockSpec((HID_DIM, HID_DIM), lambda i: (0, 0)),
            pl.BlockSpec((1, HID_DIM), lambda i: (0, 0)),
            pl.BlockSpec((HID_DIM, OUT_DIM), lambda i: (0, 0)),
            pl.BlockSpec((1, OUT_DIM), lambda i: (0, 0)),
        ],
        out_specs=pl.BlockSpec((NUM_SEGMENTS, OUT_DIM), lambda i: (0, 0)),
        out_shape=jax.ShapeDtypeStruct((NUM_SEGMENTS, OUT_DIM), jnp.float32),
        scratch_shapes=[
            pltpu.VMEM((2, BLOCK_T, NODE_DIM), jnp.float32),
            pltpu.VMEM((NODE_DIM, HID_DIM), jnp.bfloat16),
            pltpu.VMEM((HID_DIM, HID_DIM), jnp.bfloat16),
            pltpu.VMEM((NUM_SEGMENTS, HID_DIM), jnp.float32),
            pltpu.VMEM((NUM_SEGMENTS, 128), jnp.float32),
            pltpu.SemaphoreType.DMA((2,)),
        ],
        compiler_params=pltpu.CompilerParams(
            dimension_semantics=("arbitrary",),
        ),
    )(seg, x, W1, b1, W2, b2, W3, b3)
    return out


# speedup vs baseline: 1.0002x; 1.0002x over previous
import jax
import jax.numpy as jnp
from jax.experimental import pallas as pl
from jax.experimental.pallas import tpu as pltpu

NODE_DIM = 512
HID_DIM = 2048
OUT_DIM = 512
N_TOKENS = 32768
NUM_SEGMENTS = 16

BLOCK_T = 1024
NUM_BLOCKS = N_TOKENS // BLOCK_T
N_CHUNKS = 2
CHUNK_T = BLOCK_T // N_CHUNKS


def _x_copy(x_hbm, xbuf, sem, blk, slot):
    return pltpu.make_async_copy(
        x_hbm.at[pl.ds(blk * BLOCK_T, BLOCK_T), :],
        xbuf.at[slot],
        sem.at[slot],
    )


def _mlpagg_kernel(seg_ref, x_hbm, W1_ref, b1_ref, W2_ref, b2_ref,
                   W3_ref, b3_ref, out_ref, xbuf, h1_ref, acc_ref,
                   cnt_ref, sem):
    i = pl.program_id(0)
    slot = jax.lax.rem(i, 2)

    @pl.when(i == 0)
    def _init():
        acc_ref[...] = jnp.zeros_like(acc_ref)
        cnt_ref[...] = jnp.zeros_like(cnt_ref)
        _x_copy(x_hbm, xbuf, sem, 0, 0).start()

    @pl.when(i + 1 < NUM_BLOCKS)
    def _prefetch():
        _x_copy(x_hbm, xbuf, sem, i + 1, 1 - slot).start()

    _x_copy(x_hbm, xbuf, sem, i, slot).wait()

    seg = seg_ref[0, 0, :]
    onehot = (seg[None, :] == jax.lax.broadcasted_iota(
        jnp.int32, (NUM_SEGMENTS, BLOCK_T), 0)).astype(jnp.float32)
    cnt_ref[...] += jnp.sum(onehot, axis=1, keepdims=True)

    for c in range(N_CHUNKS):
        sl = slice(c * CHUNK_T, (c + 1) * CHUNK_T)
        x = xbuf[slot, sl, :]
        h = jnp.dot(x, W1_ref[...], preferred_element_type=jnp.float32)
        h1_ref[c] = jnp.maximum(h + b1_ref[...], 0.0)

    acc = jnp.zeros((NUM_SEGMENTS, HID_DIM), dtype=jnp.float32)
    for c in range(N_CHUNKS):
        sl = slice(c * CHUNK_T, (c + 1) * CHUNK_T)
        h = jnp.dot(h1_ref[c], W2_ref[...],
                    preferred_element_type=jnp.float32)
        h = jnp.maximum(h + b2_ref[...], 0.0)
        acc = acc + jnp.dot(onehot[:, sl], h,
                            preferred_element_type=jnp.float32)
    acc_ref[...] += acc

    @pl.when(i == NUM_BLOCKS - 1)
    def _finish():
        counts = jnp.maximum(cnt_ref[:, 0:1], 1.0)
        mean = acc_ref[...] / counts
        out_ref[...] = jnp.dot(
            mean, W3_ref[...], preferred_element_type=jnp.float32) + b3_ref[...]


@jax.jit
def kernel(x, x_batch, W1, b1, W2, b2, W3, b3):
    seg = x_batch.astype(jnp.int32).reshape(NUM_BLOCKS, 1, BLOCK_T)
    b1 = b1.reshape(1, HID_DIM)
    b2 = b2.reshape(1, HID_DIM)
    b3 = b3.reshape(1, OUT_DIM)

    grid = (NUM_BLOCKS,)
    out = pl.pallas_call(
        _mlpagg_kernel,
        grid=grid,
        in_specs=[
            pl.BlockSpec((1, 1, BLOCK_T), lambda i: (i, 0, 0)),
            pl.BlockSpec(memory_space=pl.ANY),
            pl.BlockSpec((NODE_DIM, HID_DIM), lambda i: (0, 0)),
            pl.BlockSpec((1, HID_DIM), lambda i: (0, 0)),
            pl.BlockSpec((HID_DIM, HID_DIM), lambda i: (0, 0)),
            pl.BlockSpec((1, HID_DIM), lambda i: (0, 0)),
            pl.BlockSpec((HID_DIM, OUT_DIM), lambda i: (0, 0)),
            pl.BlockSpec((1, OUT_DIM), lambda i: (0, 0)),
        ],
        out_specs=pl.BlockSpec((NUM_SEGMENTS, OUT_DIM), lambda i: (0, 0)),
        out_shape=jax.ShapeDtypeStruct((NUM_SEGMENTS, OUT_DIM), jnp.float32),
        scratch_shapes=[
            pltpu.VMEM((2, BLOCK_T, NODE_DIM), jnp.float32),
            pltpu.VMEM((N_CHUNKS, CHUNK_T, HID_DIM), jnp.float32),
            pltpu.VMEM((NUM_SEGMENTS, HID_DIM), jnp.float32),
            pltpu.VMEM((NUM_SEGMENTS, 128), jnp.float32),
            pltpu.SemaphoreType.DMA((2,)),
        ],
        compiler_params=pltpu.CompilerParams(
            dimension_semantics=("arbitrary",),
        ),
    )(seg, x, W1, b1, W2, b2, W3, b3)
    return out


# speedup vs baseline: 1.0115x; 1.0114x over previous
import jax
import jax.numpy as jnp
from jax.experimental import pallas as pl
from jax.experimental.pallas import tpu as pltpu

NODE_DIM = 512
HID_DIM = 2048
OUT_DIM = 512
N_TOKENS = 32768
NUM_SEGMENTS = 16

BLOCK_T = 2048
NUM_BLOCKS = N_TOKENS // BLOCK_T
N_CHUNKS = 2
CHUNK_T = BLOCK_T // N_CHUNKS


def _mlpagg_kernel(seg_ref, x_ref, W1_ref, b1_ref, W2_ref, b2_ref,
                   W3_ref, b3_ref, out_ref, acc_ref, cnt_ref):
    i = pl.program_id(0)

    @pl.when(i == 0)
    def _init():
        acc_ref[...] = jnp.zeros_like(acc_ref)
        cnt_ref[...] = jnp.zeros_like(cnt_ref)

    seg = seg_ref[0, 0, :]
    onehot = (seg[None, :] == jax.lax.broadcasted_iota(
        jnp.int32, (NUM_SEGMENTS, BLOCK_T), 0)).astype(jnp.float32)
    cnt_ref[...] += jnp.sum(onehot, axis=1, keepdims=True)

    acc = jnp.zeros((NUM_SEGMENTS, HID_DIM), dtype=jnp.float32)
    for c in range(N_CHUNKS):
        sl = slice(c * CHUNK_T, (c + 1) * CHUNK_T)
        x = x_ref[sl, :]
        h = jnp.dot(x, W1_ref[...], preferred_element_type=jnp.float32)
        h = jnp.maximum(h + b1_ref[...], 0.0)
        h = jnp.dot(h, W2_ref[...], preferred_element_type=jnp.float32)
        h = jnp.maximum(h + b2_ref[...], 0.0)
        acc = acc + jnp.dot(onehot[:, sl], h,
                            preferred_element_type=jnp.float32)
    acc_ref[...] += acc

    @pl.when(i == NUM_BLOCKS - 1)
    def _finish():
        counts = jnp.maximum(cnt_ref[:, 0:1], 1.0)
        mean = acc_ref[...] / counts
        out_ref[...] = jnp.dot(
            mean, W3_ref[...], preferred_element_type=jnp.float32) + b3_ref[...]


@jax.jit
def kernel(x, x_batch, W1, b1, W2, b2, W3, b3):
    seg = x_batch.astype(jnp.int32).reshape(NUM_BLOCKS, 1, BLOCK_T)
    b1 = b1.reshape(1, HID_DIM)
    b2 = b2.reshape(1, HID_DIM)
    b3 = b3.reshape(1, OUT_DIM)

    grid = (NUM_BLOCKS,)
    out = pl.pallas_call(
        _mlpagg_kernel,
        grid=grid,
        in_specs=[
            pl.BlockSpec((1, 1, BLOCK_T), lambda i: (i, 0, 0)),
            pl.BlockSpec((BLOCK_T, NODE_DIM), lambda i: (i, 0)),
            pl.BlockSpec((NODE_DIM, HID_DIM), lambda i: (0, 0)),
            pl.BlockSpec((1, HID_DIM), lambda i: (0, 0)),
            pl.BlockSpec((HID_DIM, HID_DIM), lambda i: (0, 0)),
            pl.BlockSpec((1, HID_DIM), lambda i: (0, 0)),
            pl.BlockSpec((HID_DIM, OUT_DIM), lambda i: (0, 0)),
            pl.BlockSpec((1, OUT_DIM), lambda i: (0, 0)),
        ],
        out_specs=pl.BlockSpec((NUM_SEGMENTS, OUT_DIM), lambda i: (0, 0)),
        out_shape=jax.ShapeDtypeStruct((NUM_SEGMENTS, OUT_DIM), jnp.float32),
        scratch_shapes=[
            pltpu.VMEM((NUM_SEGMENTS, HID_DIM), jnp.float32),
            pltpu.VMEM((NUM_SEGMENTS, 128), jnp.float32),
        ],
        compiler_params=pltpu.CompilerParams(
            dimension_semantics=("arbitrary",),
        ),
    )(seg, x, W1, b1, W2, b2, W3, b3)
    return out
